# baseline (speedup 1.0000x reference)
import functools

import jax
import jax.numpy as jnp
from jax import lax
from jax.experimental import pallas as pl
from jax.experimental.pallas import tpu as pltpu
from jax.experimental.pallas import tpu_sc as plsc

D = 64
H = 128
MAX_CAP, MAX_DEG, MAX_UNUSED, MAX_CONF = 11, 16, 9, 2
NPROD = MAX_CAP * MAX_DEG * MAX_UNUSED * MAX_CONF

NC, NS, L = 2, 16, 16
NW = NC * NS
CHUNK = 112
EPS = 1e-5


def _prep_body(capT, degT, unuT, cfT, cw, cb, w1, b1, lnw, lnb,
               tbl_ref, a_ref, b_ref, aux_ref):
    f32 = jnp.float32
    wa = w1[0:D, :]
    wb = w1[D:2 * D, :]
    wc = w1[2 * D:3 * D, :]
    wd = w1[3 * D:4 * D, :]
    we = w1[4 * D:5 * D, :]
    tc = jnp.dot(capT[...], wa, preferred_element_type=f32)
    td = jnp.dot(degT[...], wb, preferred_element_type=f32)
    tu = jnp.dot(unuT[...], wc, preferred_element_type=f32)
    tf = jnp.dot(cfT[...], wd, preferred_element_type=f32)
    r = jnp.dot(cw[...], we, preferred_element_type=f32)
    c0 = jnp.dot(cb[...], we, preferred_element_type=f32) + b1[...]

    kk = lax.broadcasted_iota(jnp.int32, (NPROD, 16), 0)
    col = lax.broadcasted_iota(jnp.int32, (NPROD, 16), 1)
    rad_f = MAX_CONF
    rad_u = MAX_UNUSED * rad_f
    rad_d = MAX_DEG * rad_u
    c_idx = kk // rad_d
    d_idx = (kk // rad_u) % MAX_DEG
    u_idx = (kk // rad_f) % MAX_UNUSED
    f_idx = kk % MAX_CONF
    t4 = (jnp.dot((col == c_idx).astype(f32), tc, preferred_element_type=f32)
          + jnp.dot((col == d_idx).astype(f32), td, preferred_element_type=f32)
          + jnp.dot((col == u_idx).astype(f32), tu, preferred_element_type=f32)
          + jnp.dot((col == f_idx).astype(f32), tf, preferred_element_type=f32)
          + c0)
    mu = jnp.mean(t4, axis=1, keepdims=True)
    tcen = t4 - mu
    rc = r - jnp.mean(r)
    tbl_ref[...] = tcen * lnw[...]
    a_ref[...] = jnp.mean(tcen * tcen, axis=1, keepdims=True)
    b_ref[...] = 2.0 * jnp.mean(tcen * rc, axis=1, keepdims=True)
    c_var = jnp.mean(rc * rc)
    row4 = lax.broadcasted_iota(jnp.int32, (4, H), 0)
    aux_ref[...] = (jnp.where(row4 == 0, rc * lnw[...], 0.0)
                    + jnp.where(row4 == 1, lnb[...], 0.0)
                    + jnp.where(row4 == 2, c_var, 0.0))


def _sc_body(n_chunks,
             xt_hbm, tbl_hbm, ta_hbm, tb_hbm, aux_hbm, out_hbm,
             capv, degv, unuv, cnfv, clov, idxv,
             invv0, invv1, gath0, gath1, av0, av1, bv0, bv1,
             auxv, outv0, outv1, semg0, semg1, semo0, semo1, semx):
    rpw = n_chunks * CHUNK
    wid = lax.axis_index("s") * NC + lax.axis_index("c")
    base0 = pl.multiple_of(wid * rpw, 8)
    pltpu.sync_copy(aux_hbm, auxv)
    rlw = [auxv[0, pl.ds(j * L, L)] for j in range(H // L)]
    lnb = [auxv[1, pl.ds(j * L, L)] for j in range(H // L)]
    cvec = auxv[2, pl.ds(0, L)]

    cps = [pltpu.async_copy(xt_hbm.at[0, pl.ds(base0, rpw)], capv, semx),
           pltpu.async_copy(xt_hbm.at[1, pl.ds(base0, rpw)], degv, semx),
           pltpu.async_copy(xt_hbm.at[2, pl.ds(base0, rpw)], unuv, semx),
           pltpu.async_copy(xt_hbm.at[3, pl.ds(base0, rpw)], cnfv, semx),
           pltpu.async_copy(xt_hbm.at[4, pl.ds(base0, rpw)],
                            clov.at[pl.ds(0, rpw)], semx)]
    for c in cps:
        c.wait()

    @plsc.parallel_loop(0, rpw, L, unroll=4)
    def idx_body(i):
        sl = pl.ds(i, L)
        idxv[sl] = ((capv[sl].astype(jnp.int32) * MAX_DEG
                     + degv[sl].astype(jnp.int32)) * MAX_UNUSED
                    + unuv[sl].astype(jnp.int32)) * MAX_CONF \
            + cnfv[sl].astype(jnp.int32)

    bufs = ((gath0, av0, bv0, invv0, outv0, semg0, semo0),
            (gath1, av1, bv1, invv1, outv1, semg1, semo1))

    def fire(ci, b):
        gath, av, bv, _, _, semg, _ = bufs[b]
        isl = idxv.at[pl.ds(ci * CHUNK, CHUNK)]
        pltpu.async_copy(tbl_hbm.at[isl], gath, semg)
        pltpu.async_copy(ta_hbm.at[isl], av, semg)
        pltpu.async_copy(tb_hbm.at[isl], bv, semg)

    def drain(ci, b):
        gath, av, bv, _, _, semg, _ = bufs[b]
        isl = idxv.at[pl.ds(ci * CHUNK, CHUNK)]
        pltpu.make_async_copy(tbl_hbm.at[isl], gath, semg).wait()
        pltpu.make_async_copy(ta_hbm.at[isl], av, semg).wait()
        pltpu.make_async_copy(tb_hbm.at[isl], bv, semg).wait()

    def out_wait(ci, b):
        _, _, _, _, outv, _, semo = bufs[b]
        base = pl.multiple_of(base0 + ci * CHUNK, 8)
        pltpu.make_async_copy(outv, out_hbm.at[pl.ds(base, CHUNK)],
                              semo).wait()

    def process(ci, b):
        gath, av, bv, invv, outv, _, semo = bufs[b]
        drain(ci, b)
        off = ci * CHUNK
        for g in range(CHUNK // L):
            sl = pl.ds(g * L, L)
            clo = clov[pl.ds(off + g * L, L)]
            var = av[sl] + clo * (bv[sl] + clo * cvec) + EPS
            ii = lax.bitcast_convert_type(var, jnp.int32)
            ii = jnp.int32(0x5F3759DF) - (ii >> 1)
            y = lax.bitcast_convert_type(ii, jnp.float32)
            for _ in range(3):
                y = y * (1.5 - 0.5 * var * y * y)
            invv[sl] = y

        @plsc.parallel_loop(0, CHUNK, 1, unroll=8)
        def row_body(i):
            clo_v = jnp.full((L,), clov[pl.ds(off + i, L)][0], jnp.float32)
            inv_v = jnp.full((L,), invv[pl.ds(i, L)][0], jnp.float32)
            for j in range(H // L):
                t = gath[i, pl.ds(j * L, L)]
                o = jnp.maximum((t + clo_v * rlw[j]) * inv_v + lnb[j], 0.0)
                outv[i, pl.ds(j * L, L)] = o

        base = pl.multiple_of(base0 + ci * CHUNK, 8)
        pltpu.async_copy(outv, out_hbm.at[pl.ds(base, CHUNK)], semo)

    fire(0, 0)

    def pair_body(m, _):
        for b in range(2):
            ci = m * 2 + b
            nxt = ci + 1

            @pl.when(nxt < n_chunks)
            def _():
                fire(nxt, 1 - b)

            @pl.when(ci >= 2)
            def _():
                out_wait(ci - 2, b)

            process(ci, b)
        return 0

    lax.fori_loop(0, n_chunks // 2, pair_body, 0)
    out_wait(n_chunks - 2, 0)
    out_wait(n_chunks - 1, 1)


def kernel(x, cap_table, deg_table, unused_table, conflict_table,
           close_W, close_b, W1, b1, ln_w, ln_b):
    f32 = jnp.float32
    n = x.shape[0]

    def pad16(t):
        return jnp.pad(t, ((0, 16 - t.shape[0]), (0, 0)))

    prep = pl.pallas_call(
        _prep_body,
        out_shape=[
            jax.ShapeDtypeStruct((NPROD, H), f32),
            jax.ShapeDtypeStruct((NPROD, 1), f32),
            jax.ShapeDtypeStruct((NPROD, 1), f32),
            jax.ShapeDtypeStruct((4, H), f32),
        ],
    )
    tbl, a2, b2, aux = prep(
        pad16(cap_table), pad16(deg_table), pad16(unused_table),
        pad16(conflict_table), close_W, close_b.reshape(1, D), W1,
        b1.reshape(1, H), ln_w.reshape(1, H), ln_b.reshape(1, H))
    ta = a2.reshape(NPROD)
    tb = b2.reshape(NPROD)

    per_round = CHUNK * NW
    n_chunks = -(-n // per_round)
    n_chunks += n_chunks % 2
    npad = n_chunks * per_round
    rpw = n_chunks * CHUNK
    xt = jnp.pad(x.T, ((0, 0), (0, npad - n)))

    mesh = plsc.VectorSubcoreMesh(core_axis_name="c", subcore_axis_name="s")
    sc = pl.kernel(
        functools.partial(_sc_body, n_chunks),
        out_type=jax.ShapeDtypeStruct((npad, H), f32),
        mesh=mesh,
        compiler_params=pltpu.CompilerParams(use_tc_tiling_on_sc=False),
        scratch_types=[
            pltpu.VMEM((rpw,), f32),
            pltpu.VMEM((rpw,), f32),
            pltpu.VMEM((rpw,), f32),
            pltpu.VMEM((rpw,), f32),
            pltpu.VMEM((rpw + L,), f32),
            pltpu.VMEM((rpw,), jnp.int32),
            pltpu.VMEM((CHUNK + L,), f32),
            pltpu.VMEM((CHUNK + L,), f32),
            pltpu.VMEM((CHUNK, H), f32),
            pltpu.VMEM((CHUNK, H), f32),
            pltpu.VMEM((CHUNK,), f32),
            pltpu.VMEM((CHUNK,), f32),
            pltpu.VMEM((CHUNK,), f32),
            pltpu.VMEM((CHUNK,), f32),
            pltpu.VMEM((4, H), f32),
            pltpu.VMEM((CHUNK, H), f32),
            pltpu.VMEM((CHUNK, H), f32),
            pltpu.SemaphoreType.DMA,
            pltpu.SemaphoreType.DMA,
            pltpu.SemaphoreType.DMA,
            pltpu.SemaphoreType.DMA,
            pltpu.SemaphoreType.DMA,
        ],
    )
    out = sc(xt, tbl, ta, tb, aux)
    return out[:n]


# speedup vs baseline: 1.0034x; 1.0034x over previous
import functools

import jax
import jax.numpy as jnp
from jax import lax
from jax.experimental import pallas as pl
from jax.experimental.pallas import tpu as pltpu
from jax.experimental.pallas import tpu_sc as plsc

D = 64
H = 128
MAX_CAP, MAX_DEG, MAX_UNUSED, MAX_CONF = 11, 16, 9, 2
NPROD = MAX_CAP * MAX_DEG * MAX_UNUSED * MAX_CONF

NC, NS, L = 2, 16, 16
NW = NC * NS
CHUNK = 112
EPS = 1e-5


def _prep_body(capT, degT, unuT, cfT, cw, cb, w1, b1, lnw, lnb,
               tbl_ref, a_ref, b_ref, aux_ref):
    f32 = jnp.float32
    wa = w1[0:D, :]
    wb = w1[D:2 * D, :]
    wc = w1[2 * D:3 * D, :]
    wd = w1[3 * D:4 * D, :]
    we = w1[4 * D:5 * D, :]
    tc = jnp.dot(capT[...], wa, preferred_element_type=f32)
    td = jnp.dot(degT[...], wb, preferred_element_type=f32)
    tu = jnp.dot(unuT[...], wc, preferred_element_type=f32)
    tf = jnp.dot(cfT[...], wd, preferred_element_type=f32)
    r = jnp.dot(cw[...], we, preferred_element_type=f32)
    c0 = jnp.dot(cb[...], we, preferred_element_type=f32) + b1[...]

    kk = lax.broadcasted_iota(jnp.int32, (NPROD, 16), 0)
    col = lax.broadcasted_iota(jnp.int32, (NPROD, 16), 1)
    rad_f = MAX_CONF
    rad_u = MAX_UNUSED * rad_f
    rad_d = MAX_DEG * rad_u
    c_idx = kk // rad_d
    d_idx = (kk // rad_u) % MAX_DEG
    u_idx = (kk // rad_f) % MAX_UNUSED
    f_idx = kk % MAX_CONF
    t4 = (jnp.dot((col == c_idx).astype(f32), tc, preferred_element_type=f32)
          + jnp.dot((col == d_idx).astype(f32), td, preferred_element_type=f32)
          + jnp.dot((col == u_idx).astype(f32), tu, preferred_element_type=f32)
          + jnp.dot((col == f_idx).astype(f32), tf, preferred_element_type=f32)
          + c0)
    mu = jnp.mean(t4, axis=1, keepdims=True)
    tcen = t4 - mu
    rc = r - jnp.mean(r)
    tbl_ref[...] = tcen * lnw[...]
    a_ref[...] = jnp.mean(tcen * tcen, axis=1, keepdims=True)
    b_ref[...] = 2.0 * jnp.mean(tcen * rc, axis=1, keepdims=True)
    c_var = jnp.mean(rc * rc)
    row4 = lax.broadcasted_iota(jnp.int32, (4, H), 0)
    aux_ref[...] = (jnp.where(row4 == 0, rc * lnw[...], 0.0)
                    + jnp.where(row4 == 1, lnb[...], 0.0)
                    + jnp.where(row4 == 2, c_var, 0.0))


def _sc_body(n_chunks,
             xt_hbm, tbl_hbm, ta_hbm, tb_hbm, aux_hbm, out_hbm,
             capv, degv, unuv, cnfv, clov, idxv,
             invv0, invv1, gath0, gath1, av0, av1, bv0, bv1,
             auxv, outv0, outv1, semg0, semg1, semo0, semo1, semx):
    rpw = n_chunks * CHUNK
    wid = lax.axis_index("s") * NC + lax.axis_index("c")
    base0 = pl.multiple_of(wid * rpw, 8)
    pltpu.sync_copy(aux_hbm, auxv)
    rlw = [auxv[0, pl.ds(j * L, L)] for j in range(H // L)]
    lnb = [auxv[1, pl.ds(j * L, L)] for j in range(H // L)]
    cvec = auxv[2, pl.ds(0, L)]

    cps = [pltpu.async_copy(xt_hbm.at[0, pl.ds(base0, rpw)], capv, semx),
           pltpu.async_copy(xt_hbm.at[1, pl.ds(base0, rpw)], degv, semx),
           pltpu.async_copy(xt_hbm.at[2, pl.ds(base0, rpw)], unuv, semx),
           pltpu.async_copy(xt_hbm.at[3, pl.ds(base0, rpw)], cnfv, semx),
           pltpu.async_copy(xt_hbm.at[4, pl.ds(base0, rpw)],
                            clov.at[pl.ds(0, rpw)], semx)]
    for c in cps:
        c.wait()

    @plsc.parallel_loop(0, rpw, L, unroll=4)
    def idx_body(i):
        sl = pl.ds(i, L)
        idxv[sl] = ((capv[sl].astype(jnp.int32) * MAX_DEG
                     + degv[sl].astype(jnp.int32)) * MAX_UNUSED
                    + unuv[sl].astype(jnp.int32)) * MAX_CONF \
            + cnfv[sl].astype(jnp.int32)

    bufs = ((gath0, av0, bv0, invv0, outv0, semg0, semo0),
            (gath1, av1, bv1, invv1, outv1, semg1, semo1))

    def fire(ci, b):
        gath, av, bv, _, _, semg, _ = bufs[b]
        isl = idxv.at[pl.ds(ci * CHUNK, CHUNK)]
        pltpu.async_copy(tbl_hbm.at[isl], gath, semg)
        pltpu.async_copy(ta_hbm.at[isl], av, semg)
        pltpu.async_copy(tb_hbm.at[isl], bv, semg)

    def drain(ci, b):
        gath, av, bv, _, _, semg, _ = bufs[b]
        isl = idxv.at[pl.ds(ci * CHUNK, CHUNK)]
        pltpu.make_async_copy(tbl_hbm.at[isl], gath, semg).wait()
        pltpu.make_async_copy(ta_hbm.at[isl], av, semg).wait()
        pltpu.make_async_copy(tb_hbm.at[isl], bv, semg).wait()

    def out_wait(ci, b):
        _, _, _, _, outv, _, semo = bufs[b]
        base = pl.multiple_of(base0 + ci * CHUNK, 8)
        pltpu.make_async_copy(outv, out_hbm.at[pl.ds(base, CHUNK)],
                              semo).wait()

    def process(ci, b):
        gath, av, bv, invv, outv, _, semo = bufs[b]
        drain(ci, b)
        off = ci * CHUNK
        for g in range(CHUNK // L):
            sl = pl.ds(g * L, L)
            clo = clov[pl.ds(off + g * L, L)]
            var = av[sl] + clo * (bv[sl] + clo * cvec) + EPS
            ii = lax.bitcast_convert_type(var, jnp.int32)
            ii = jnp.int32(0x5F3759DF) - (ii >> 1)
            y = lax.bitcast_convert_type(ii, jnp.float32)
            for _ in range(3):
                y = y * (1.5 - 0.5 * var * y * y)
            invv[sl] = y

        @plsc.parallel_loop(0, CHUNK, 1, unroll=4)
        def row_body(i):
            clo_v = jnp.full((L,), clov[pl.ds(off + i, L)][0], jnp.float32)
            inv_v = jnp.full((L,), invv[pl.ds(i, L)][0], jnp.float32)
            for j in range(H // L):
                t = gath[i, pl.ds(j * L, L)]
                o = jnp.maximum((t + clo_v * rlw[j]) * inv_v + lnb[j], 0.0)
                outv[i, pl.ds(j * L, L)] = o

        base = pl.multiple_of(base0 + ci * CHUNK, 8)
        pltpu.async_copy(outv, out_hbm.at[pl.ds(base, CHUNK)], semo)

    fire(0, 0)

    def pair_body(m, _):
        for b in range(2):
            ci = m * 2 + b
            nxt = ci + 1

            @pl.when(nxt < n_chunks)
            def _():
                fire(nxt, 1 - b)

            @pl.when(ci >= 2)
            def _():
                out_wait(ci - 2, b)

            process(ci, b)
        return 0

    lax.fori_loop(0, n_chunks // 2, pair_body, 0)
    out_wait(n_chunks - 2, 0)
    out_wait(n_chunks - 1, 1)


def kernel(x, cap_table, deg_table, unused_table, conflict_table,
           close_W, close_b, W1, b1, ln_w, ln_b):
    f32 = jnp.float32
    n = x.shape[0]

    def pad16(t):
        return jnp.pad(t, ((0, 16 - t.shape[0]), (0, 0)))

    prep = pl.pallas_call(
        _prep_body,
        out_shape=[
            jax.ShapeDtypeStruct((NPROD, H), f32),
            jax.ShapeDtypeStruct((NPROD, 1), f32),
            jax.ShapeDtypeStruct((NPROD, 1), f32),
            jax.ShapeDtypeStruct((4, H), f32),
        ],
    )
    tbl, a2, b2, aux = prep(
        pad16(cap_table), pad16(deg_table), pad16(unused_table),
        pad16(conflict_table), close_W, close_b.reshape(1, D), W1,
        b1.reshape(1, H), ln_w.reshape(1, H), ln_b.reshape(1, H))
    ta = a2.reshape(NPROD)
    tb = b2.reshape(NPROD)

    per_round = CHUNK * NW
    n_chunks = -(-n // per_round)
    n_chunks += n_chunks % 2
    npad = n_chunks * per_round
    rpw = n_chunks * CHUNK
    xt = jnp.pad(x.T, ((0, 0), (0, npad - n)))

    mesh = plsc.VectorSubcoreMesh(core_axis_name="c", subcore_axis_name="s")
    sc = pl.kernel(
        functools.partial(_sc_body, n_chunks),
        out_type=jax.ShapeDtypeStruct((npad, H), f32),
        mesh=mesh,
        compiler_params=pltpu.CompilerParams(use_tc_tiling_on_sc=False),
        scratch_types=[
            pltpu.VMEM((rpw,), f32),
            pltpu.VMEM((rpw,), f32),
            pltpu.VMEM((rpw,), f32),
            pltpu.VMEM((rpw,), f32),
            pltpu.VMEM((rpw + L,), f32),
            pltpu.VMEM((rpw,), jnp.int32),
            pltpu.VMEM((CHUNK + L,), f32),
            pltpu.VMEM((CHUNK + L,), f32),
            pltpu.VMEM((CHUNK, H), f32),
            pltpu.VMEM((CHUNK, H), f32),
            pltpu.VMEM((CHUNK,), f32),
            pltpu.VMEM((CHUNK,), f32),
            pltpu.VMEM((CHUNK,), f32),
            pltpu.VMEM((CHUNK,), f32),
            pltpu.VMEM((4, H), f32),
            pltpu.VMEM((CHUNK, H), f32),
            pltpu.VMEM((CHUNK, H), f32),
            pltpu.SemaphoreType.DMA,
            pltpu.SemaphoreType.DMA,
            pltpu.SemaphoreType.DMA,
            pltpu.SemaphoreType.DMA,
            pltpu.SemaphoreType.DMA,
        ],
    )
    out = sc(xt, tbl, ta, tb, aux)
    return out[:n]


# speedup vs baseline: 1.1524x; 1.1486x over previous
import functools

import jax
import jax.numpy as jnp
from jax import lax
from jax.experimental import pallas as pl
from jax.experimental.pallas import tpu as pltpu
from jax.experimental.pallas import tpu_sc as plsc

D = 64
H = 128
MAX_CAP, MAX_DEG, MAX_UNUSED, MAX_CONF = 11, 16, 9, 2
NPROD = MAX_CAP * MAX_DEG * MAX_UNUSED * MAX_CONF

NC, NS, L = 2, 16, 16
NW = NC * NS
CHUNK = 112
EPS = 1e-5


def _prep_body(capT, degT, unuT, cfT, cw, cb, w1, b1, lnw, lnb,
               tbl_ref, a_ref, b_ref, aux_ref):
    f32 = jnp.float32
    wa = w1[0:D, :]
    wb = w1[D:2 * D, :]
    wc = w1[2 * D:3 * D, :]
    wd = w1[3 * D:4 * D, :]
    we = w1[4 * D:5 * D, :]
    tc = jnp.dot(capT[...], wa, preferred_element_type=f32)
    td = jnp.dot(degT[...], wb, preferred_element_type=f32)
    tu = jnp.dot(unuT[...], wc, preferred_element_type=f32)
    tf = jnp.dot(cfT[...], wd, preferred_element_type=f32)
    r = jnp.dot(cw[...], we, preferred_element_type=f32)
    c0 = jnp.dot(cb[...], we, preferred_element_type=f32) + b1[...]

    kk = lax.broadcasted_iota(jnp.int32, (NPROD, 16), 0)
    col = lax.broadcasted_iota(jnp.int32, (NPROD, 16), 1)
    rad_f = MAX_CONF
    rad_u = MAX_UNUSED * rad_f
    rad_d = MAX_DEG * rad_u
    c_idx = kk // rad_d
    d_idx = (kk // rad_u) % MAX_DEG
    u_idx = (kk // rad_f) % MAX_UNUSED
    f_idx = kk % MAX_CONF
    t4 = (jnp.dot((col == c_idx).astype(f32), tc, preferred_element_type=f32)
          + jnp.dot((col == d_idx).astype(f32), td, preferred_element_type=f32)
          + jnp.dot((col == u_idx).astype(f32), tu, preferred_element_type=f32)
          + jnp.dot((col == f_idx).astype(f32), tf, preferred_element_type=f32)
          + c0)
    mu = jnp.mean(t4, axis=1, keepdims=True)
    tcen = t4 - mu
    rc = r - jnp.mean(r)
    tbl_ref[...] = tcen * lnw[...]
    a_ref[...] = jnp.mean(tcen * tcen, axis=1, keepdims=True)
    b_ref[...] = 2.0 * jnp.mean(tcen * rc, axis=1, keepdims=True)
    c_var = jnp.mean(rc * rc)
    row4 = lax.broadcasted_iota(jnp.int32, (4, H), 0)
    aux_ref[...] = (jnp.where(row4 == 0, rc * lnw[...], 0.0)
                    + jnp.where(row4 == 1, lnb[...], 0.0)
                    + jnp.where(row4 == 2, c_var, 0.0))


def _sc_body(n_chunks, n_rows,
             xt_hbm, tbl_hbm, ta_hbm, tb_hbm, aux_hbm, out_hbm,
             capv, degv, unuv, cnfv, clov, idxv,
             invv0, invv1, gath0, gath1, av0, av1, bv0, bv1,
             auxv, outv0, outv1, semg0, semg1, semo0, semo1, semx):
    rpw = n_chunks * CHUNK
    wid = lax.axis_index("s") * NC + lax.axis_index("c")
    base0 = pl.multiple_of(wid * rpw, 8)
    pltpu.sync_copy(aux_hbm, auxv)
    rlw = [auxv[0, pl.ds(j * L, L)] for j in range(H // L)]
    lnb = [auxv[1, pl.ds(j * L, L)] for j in range(H // L)]
    cvec = auxv[2, pl.ds(0, L)]

    cps = [pltpu.async_copy(xt_hbm.at[0, pl.ds(base0, rpw)], capv, semx),
           pltpu.async_copy(xt_hbm.at[1, pl.ds(base0, rpw)], degv, semx),
           pltpu.async_copy(xt_hbm.at[2, pl.ds(base0, rpw)], unuv, semx),
           pltpu.async_copy(xt_hbm.at[3, pl.ds(base0, rpw)], cnfv, semx),
           pltpu.async_copy(xt_hbm.at[4, pl.ds(base0, rpw)],
                            clov.at[pl.ds(0, rpw)], semx)]
    for c in cps:
        c.wait()

    @plsc.parallel_loop(0, rpw, L, unroll=4)
    def idx_body(i):
        sl = pl.ds(i, L)
        idxv[sl] = ((capv[sl].astype(jnp.int32) * MAX_DEG
                     + degv[sl].astype(jnp.int32)) * MAX_UNUSED
                    + unuv[sl].astype(jnp.int32)) * MAX_CONF \
            + cnfv[sl].astype(jnp.int32)

    bufs = ((gath0, av0, bv0, invv0, outv0, semg0, semo0),
            (gath1, av1, bv1, invv1, outv1, semg1, semo1))

    def fire(ci, b):
        gath, av, bv, _, _, semg, _ = bufs[b]
        isl = idxv.at[pl.ds(ci * CHUNK, CHUNK)]
        pltpu.async_copy(tbl_hbm.at[isl], gath, semg)
        pltpu.async_copy(ta_hbm.at[isl], av, semg)
        pltpu.async_copy(tb_hbm.at[isl], bv, semg)

    def drain(ci, b):
        gath, av, bv, _, _, semg, _ = bufs[b]
        isl = idxv.at[pl.ds(ci * CHUNK, CHUNK)]
        pltpu.make_async_copy(tbl_hbm.at[isl], gath, semg).wait()
        pltpu.make_async_copy(ta_hbm.at[isl], av, semg).wait()
        pltpu.make_async_copy(tb_hbm.at[isl], bv, semg).wait()

    def out_wait(ci, b):
        _, _, _, _, outv, _, semo = bufs[b]
        base = pl.multiple_of(base0 + ci * CHUNK, 8)

        @pl.when(base + CHUNK <= n_rows)
        def _():
            pltpu.make_async_copy(outv, out_hbm.at[pl.ds(base, CHUNK)],
                                  semo).wait()

    def out_store(ci, b, outv, semo):
        base = pl.multiple_of(base0 + ci * CHUNK, 8)

        @pl.when(base + CHUNK <= n_rows)
        def _():
            pltpu.async_copy(outv, out_hbm.at[pl.ds(base, CHUNK)], semo)

        @pl.when(base + CHUNK > n_rows)
        def _():
            for kk in range(CHUNK // L):
                @pl.when(base + (kk + 1) * L <= n_rows)
                def _():
                    pltpu.sync_copy(
                        outv.at[pl.ds(kk * L, L)],
                        out_hbm.at[pl.ds(pl.multiple_of(base + kk * L, 8),
                                         L)])

    def process(ci, b):
        gath, av, bv, invv, outv, _, semo = bufs[b]
        drain(ci, b)
        off = ci * CHUNK
        for g in range(CHUNK // L):
            sl = pl.ds(g * L, L)
            clo = clov[pl.ds(off + g * L, L)]
            var = av[sl] + clo * (bv[sl] + clo * cvec) + EPS
            ii = lax.bitcast_convert_type(var, jnp.int32)
            ii = jnp.int32(0x5F3759DF) - (ii >> 1)
            y = lax.bitcast_convert_type(ii, jnp.float32)
            for _ in range(3):
                y = y * (1.5 - 0.5 * var * y * y)
            invv[sl] = y

        @plsc.parallel_loop(0, CHUNK, 1, unroll=4)
        def row_body(i):
            clo_v = jnp.full((L,), clov[pl.ds(off + i, L)][0], jnp.float32)
            inv_v = jnp.full((L,), invv[pl.ds(i, L)][0], jnp.float32)
            for j in range(H // L):
                t = gath[i, pl.ds(j * L, L)]
                o = jnp.maximum((t + clo_v * rlw[j]) * inv_v + lnb[j], 0.0)
                outv[i, pl.ds(j * L, L)] = o

        out_store(ci, b, outv, semo)

    fire(0, 0)

    def pair_body(m, _):
        for b in range(2):
            ci = m * 2 + b
            nxt = ci + 1

            @pl.when(nxt < n_chunks)
            def _():
                fire(nxt, 1 - b)

            @pl.when(ci >= 2)
            def _():
                out_wait(ci - 2, b)

            process(ci, b)
        return 0

    lax.fori_loop(0, n_chunks // 2, pair_body, 0)
    out_wait(n_chunks - 2, 0)
    out_wait(n_chunks - 1, 1)


def kernel(x, cap_table, deg_table, unused_table, conflict_table,
           close_W, close_b, W1, b1, ln_w, ln_b):
    f32 = jnp.float32
    n = x.shape[0]

    def pad16(t):
        return jnp.pad(t, ((0, 16 - t.shape[0]), (0, 0)))

    prep = pl.pallas_call(
        _prep_body,
        out_shape=[
            jax.ShapeDtypeStruct((NPROD, H), f32),
            jax.ShapeDtypeStruct((NPROD, 1), f32),
            jax.ShapeDtypeStruct((NPROD, 1), f32),
            jax.ShapeDtypeStruct((4, H), f32),
        ],
    )
    tbl, a2, b2, aux = prep(
        pad16(cap_table), pad16(deg_table), pad16(unused_table),
        pad16(conflict_table), close_W, close_b.reshape(1, D), W1,
        b1.reshape(1, H), ln_w.reshape(1, H), ln_b.reshape(1, H))
    ta = a2.reshape(NPROD)
    tb = b2.reshape(NPROD)

    per_round = CHUNK * NW
    n_chunks = -(-n // per_round)
    n_chunks += n_chunks % 2
    npad = n_chunks * per_round
    rpw = n_chunks * CHUNK
    xt = jnp.pad(x.T, ((0, 0), (0, npad - n)))
    n_out = n if n % L == 0 else npad

    mesh = plsc.VectorSubcoreMesh(core_axis_name="c", subcore_axis_name="s")
    sc = pl.kernel(
        functools.partial(_sc_body, n_chunks, n_out),
        out_type=jax.ShapeDtypeStruct((n_out, H), f32),
        mesh=mesh,
        compiler_params=pltpu.CompilerParams(use_tc_tiling_on_sc=False),
        scratch_types=[
            pltpu.VMEM((rpw,), f32),
            pltpu.VMEM((rpw,), f32),
            pltpu.VMEM((rpw,), f32),
            pltpu.VMEM((rpw,), f32),
            pltpu.VMEM((rpw + L,), f32),
            pltpu.VMEM((rpw,), jnp.int32),
            pltpu.VMEM((CHUNK + L,), f32),
            pltpu.VMEM((CHUNK + L,), f32),
            pltpu.VMEM((CHUNK, H), f32),
            pltpu.VMEM((CHUNK, H), f32),
            pltpu.VMEM((CHUNK,), f32),
            pltpu.VMEM((CHUNK,), f32),
            pltpu.VMEM((CHUNK,), f32),
            pltpu.VMEM((CHUNK,), f32),
            pltpu.VMEM((4, H), f32),
            pltpu.VMEM((CHUNK, H), f32),
            pltpu.VMEM((CHUNK, H), f32),
            pltpu.SemaphoreType.DMA,
            pltpu.SemaphoreType.DMA,
            pltpu.SemaphoreType.DMA,
            pltpu.SemaphoreType.DMA,
            pltpu.SemaphoreType.DMA,
        ],
    )
    out = sc(xt, tbl, ta, tb, aux)
    return out if n_out == n else out[:n]
